# Initial kernel scaffold; baseline (speedup 1.0000x reference)
import jax
import jax.numpy as jnp
from jax.experimental import pallas as pl


def kernel(inputs, codebook):
    raise NotImplementedError("write your pallas kernel here")


# speedup vs baseline: 1.2526x; 1.2526x over previous
import jax
import jax.numpy as jnp
from jax.experimental import pallas as pl
from jax.experimental.pallas import tpu as pltpu
from jax.experimental.pallas import tpu_sc as plsc

_K = 8192
_D = 32
_N = 16384
_BR = 256
_GRID = _N // _BR
_BETA = 0.25
_WIN = 128


def _tc_body(x_ref, fsq_ref, csq_ref, cb_ref, idx_ref, loss_ref):
    i = pl.program_id(0)
    x = x_ref[...]
    cb = cb_ref[...]
    m = jax.lax.dot_general(x, cb, (((1,), (1,)), ((), ())),
                            preferred_element_type=jnp.float32)
    d = (fsq_ref[...] - 2 * m) + csq_ref[...]
    mind = jnp.min(d, axis=1, keepdims=True)
    iot = jax.lax.broadcasted_iota(jnp.int32, (_BR, _K), 1)
    idx = jnp.min(jnp.where(d == mind, iot, _K), axis=1)
    idx_ref[0, 0, :] = idx

    @pl.when(i == 0)
    def _():
        loss_ref[...] = jnp.zeros((1, 1), jnp.float32)

    loss_ref[...] += jnp.sum(mind, keepdims=True)

    @pl.when(i == _GRID - 1)
    def _():
        loss_ref[...] = loss_ref[...] * ((1.0 + _BETA) / (_N * _D))


def _tc_argmin(xf, fsq, csq, cb):
    return pl.pallas_call(
        _tc_body,
        grid=(_GRID,),
        in_specs=[
            pl.BlockSpec((_BR, _D), lambda i: (i, 0)),
            pl.BlockSpec((_BR, 1), lambda i: (i, 0)),
            pl.BlockSpec((1, _K), lambda i: (0, 0)),
            pl.BlockSpec((_K, _D), lambda i: (0, 0)),
        ],
        out_specs=[
            pl.BlockSpec((1, 1, _BR), lambda i: (i, 0, 0)),
            pl.BlockSpec((1, 1), lambda i: (0, 0)),
        ],
        out_shape=[
            jax.ShapeDtypeStruct((_GRID, 1, _BR), jnp.int32),
            jax.ShapeDtypeStruct((1, 1), jnp.float32),
        ],
    )(xf, fsq, csq, cb)


def _sc_gather(cbp, idx2d):
    @pl.kernel(
        out_type=jax.ShapeDtypeStruct((_N, 128), jnp.float32),
        mesh=plsc.VectorSubcoreMesh(core_axis_name="core",
                                    subcore_axis_name="subcore"),
    )
    def k(cb_hbm, i_hbm, o_hbm):
        def body(i_vmem, o_vmem):
            pltpu.sync_copy(cb_hbm.at[i_vmem.at[0]], o_vmem)

        pltpu.emit_pipeline(
            body,
            grid=(_N // _WIN,),
            in_specs=[pl.BlockSpec((1, _WIN), index_map=lambda i: (0, i))],
            out_specs=[pl.BlockSpec((_WIN, 128), index_map=lambda i: (i, 0))],
            core_axis_name=("core", "subcore"),
            dimension_semantics=(pltpu.PARALLEL,),
        )(i_hbm, o_hbm)

    return k(cbp, idx2d)


def kernel(inputs, codebook):
    xf = inputs.reshape(-1, _D)
    fsq = jnp.sum(xf ** 2, axis=-1, keepdims=True)
    csq = jnp.sum(codebook ** 2, axis=-1, keepdims=True).T
    idx_blocks, loss = _tc_argmin(xf, fsq, csq, codebook)
    idx = idx_blocks.reshape(_N)
    cbp = jnp.pad(codebook, ((0, 0), (0, 128 - _D)))
    q = _sc_gather(cbp, idx.reshape(1, _N))[:, :_D].reshape(inputs.shape)
    quantize = inputs + jax.lax.stop_gradient(q - inputs)
    encoding_indices = idx.reshape(inputs.shape[:-1])
    return loss.reshape(()), quantize, encoding_indices
